# Initial kernel scaffold; baseline (speedup 1.0000x reference)
import jax
import jax.numpy as jnp
from jax.experimental import pallas as pl


def kernel(words, table):
    raise NotImplementedError("write your pallas kernel here")


# speedup vs baseline: 2.6223x; 2.6223x over previous
import functools

import jax
import jax.numpy as jnp
import numpy as np
from jax import lax
from jax.experimental import pallas as pl
from jax.experimental.pallas import tpu as pltpu
from jax.experimental.pallas import tpu_sc as plsc

_VOCAB = 100000
_DIM = 128
_SCALE = float(np.sqrt(np.float32(_DIM)))

_B = 4096 * 50
_NC, _NS, _L = 2, 16, 16
_NW = _NC * _NS
_BPW = _B // _NW
_CHUNK = 400
_NCHUNK = _BPW // _CHUNK

_mesh = plsc.VectorSubcoreMesh(core_axis_name="c", subcore_axis_name="s")


@functools.partial(
    pl.kernel,
    mesh=_mesh,
    out_type=jax.ShapeDtypeStruct((_B, _DIM), jnp.float32),
    scratch_types=[
        pltpu.VMEM((_BPW,), jnp.int32),
        pltpu.VMEM((_CHUNK, _DIM), jnp.float32),
        pltpu.SemaphoreType.DMA,
    ],
)
def _emb_lookup(words_hbm, table_hbm, out_hbm, idx_v, rows_v, gsem):
    wid = lax.axis_index("s") * _NC + lax.axis_index("c")
    base = wid * _BPW
    pltpu.sync_copy(words_hbm.at[pl.ds(base, _BPW)], idx_v)

    def chunk_body(c, carry):
        off = pl.multiple_of(c * _CHUNK, 8)
        pltpu.async_copy(
            table_hbm.at[idx_v.at[pl.ds(off, _CHUNK)]], rows_v, gsem
        ).wait()

        def scale_body(r, carry2):
            for j in range(_DIM // _L):
                sl = pl.ds(j * _L, _L)
                rows_v[r, sl] = rows_v[r, sl] * _SCALE
            return carry2

        lax.fori_loop(0, _CHUNK, scale_body, 0)
        pltpu.sync_copy(rows_v, out_hbm.at[pl.ds(base + off, _CHUNK)])
        return carry

    lax.fori_loop(0, _NCHUNK, chunk_body, 0)


def kernel(words, table):
    flat = words.reshape(-1).astype(jnp.int32)
    out = _emb_lookup(flat, table)
    return out.reshape(words.shape + (_DIM,))


# speedup vs baseline: 2.9156x; 1.1119x over previous
import functools

import jax
import jax.numpy as jnp
import numpy as np
from jax import lax
from jax.experimental import pallas as pl
from jax.experimental.pallas import tpu as pltpu
from jax.experimental.pallas import tpu_sc as plsc

_VOCAB = 100000
_DIM = 128
_SCALE = float(np.sqrt(np.float32(_DIM)))

_B = 4096 * 50
_NC, _NS, _L = 2, 16, 16
_NW = _NC * _NS
_BPW = _B // _NW
_CHUNK = 400
_NCHUNK = _BPW // _CHUNK

_mesh = plsc.VectorSubcoreMesh(core_axis_name="c", subcore_axis_name="s")


@functools.partial(
    pl.kernel,
    mesh=_mesh,
    out_type=jax.ShapeDtypeStruct((_B, _DIM), jnp.float32),
    scratch_types=[
        pltpu.VMEM((_BPW,), jnp.int32),
        pltpu.VMEM((_CHUNK, _DIM), jnp.float32),
        pltpu.VMEM((_CHUNK, _DIM), jnp.float32),
        pltpu.SemaphoreType.DMA,
        pltpu.SemaphoreType.DMA,
        pltpu.SemaphoreType.DMA,
        pltpu.SemaphoreType.DMA,
    ],
)
def _emb_lookup(words_hbm, table_hbm, out_hbm, idx_v, buf0, buf1, g0, g1, s0, s1):
    wid = lax.axis_index("s") * _NC + lax.axis_index("c")
    base = wid * _BPW
    pltpu.sync_copy(words_hbm.at[pl.ds(base, _BPW)], idx_v)

    bufs = (buf0, buf1)
    gsems = (g0, g1)
    ssems = (s0, s1)

    def start_gather(c):
        off = c * _CHUNK
        return pltpu.async_copy(
            table_hbm.at[idx_v.at[pl.ds(off, _CHUNK)]], bufs[c % 2], gsems[c % 2]
        )

    def scale(buf):
        def scale_body(r, carry):
            for j in range(_DIM // _L):
                sl = pl.ds(j * _L, _L)
                buf[r, sl] = buf[r, sl] * _SCALE
            return carry

        lax.fori_loop(0, _CHUNK, scale_body, 0)

    gathers = [None] * _NCHUNK
    scatters = [None] * _NCHUNK
    gathers[0] = start_gather(0)
    for c in range(_NCHUNK):
        cb = c % 2
        if c + 1 < _NCHUNK:
            if c >= 1:
                scatters[c - 1].wait()
            gathers[c + 1] = start_gather(c + 1)
        gathers[c].wait()
        scale(bufs[cb])
        scatters[c] = pltpu.async_copy(
            bufs[cb], out_hbm.at[pl.ds(base + c * _CHUNK, _CHUNK)], ssems[cb]
        )
    scatters[_NCHUNK - 2].wait()
    scatters[_NCHUNK - 1].wait()


def kernel(words, table):
    flat = words.reshape(-1).astype(jnp.int32)
    out = _emb_lookup(flat, table)
    return out.reshape(words.shape + (_DIM,))
